# baseline (speedup 1.0000x reference)
import functools

import jax
import jax.numpy as jnp
from jax import lax
from jax.experimental import pallas as pl
from jax.experimental.pallas import tpu as pltpu
from jax.experimental.pallas import tpu_sc as plsc

N_NODES = 10000
N_EDGES = 320000
D = 128

NC = 2
NS = 16
NW = NC * NS
K = 128
CHUNKS = 80
EPT = CHUNKS * K
HALFC = CHUNKS // 2
QC = CHUNKS // 5
E_PAD = NW * EPT
N_PAD = 10240
ROWS_PER_TILE = N_PAD // NS


def _sc_bincount_body(src_hbm, dst_hbm, deg_hbm, src_v, dst_v, ones_v, zero_v,
                      deg_src_sh, deg_dst_sh, bsem_s, bsem_d):
    c = lax.axis_index("c")
    s = lax.axis_index("s")
    wid = c * NS + s

    for i in range(ROWS_PER_TILE // 16):
        zero_v[pl.ds(i * 16, 16)] = jnp.zeros((16,), jnp.float32)
    for i in range(K // 16):
        ones_v[pl.ds(i * 16, 16)] = jnp.ones((16,), jnp.float32)
    pltpu.sync_copy(zero_v, deg_src_sh.at[pl.ds(s * ROWS_PER_TILE, ROWS_PER_TILE)])
    pltpu.sync_copy(zero_v, deg_dst_sh.at[pl.ds(s * ROWS_PER_TILE, ROWS_PER_TILE)])

    plsc.subcore_barrier()

    def body(j, carry):
        pltpu.async_copy(ones_v, deg_src_sh.at[src_v.at[j]], bsem_s, add=True)
        pltpu.async_copy(ones_v, deg_dst_sh.at[dst_v.at[j]], bsem_d, add=True)
        pltpu.make_async_copy(ones_v, deg_src_sh.at[src_v.at[j]], bsem_s).wait()
        pltpu.make_async_copy(ones_v, deg_dst_sh.at[dst_v.at[j]], bsem_d).wait()
        return carry

    for phase in range(2):
        pltpu.sync_copy(src_hbm.at[wid, pl.ds(phase * HALFC, HALFC)], src_v)
        pltpu.sync_copy(dst_hbm.at[wid, pl.ds(phase * HALFC, HALFC)], dst_v)
        lax.fori_loop(0, HALFC, body, 0)
    plsc.subcore_barrier()

    sl = pl.ds(s * ROWS_PER_TILE, ROWS_PER_TILE)
    pltpu.sync_copy(deg_src_sh.at[sl], deg_hbm.at[c, 0, sl])
    pltpu.sync_copy(deg_dst_sh.at[sl], deg_hbm.at[c, 1, sl])


def _sc_scatter_body(xs_hbm, src_hbm, dst_hbm, out_hbm, src_q0, dst_q0,
                     src_q1, dst_q1, rows_a, rows_b, acc_sh,
                     gsem_a, gsem_b, ssem_a, ssem_b, isem_s, isem_d):
    c = lax.axis_index("c")
    s = lax.axis_index("s")
    wid = c * NS + s

    def zrow(i, carry):
        for t in range(D // 16):
            rows_a[i, pl.ds(t * 16, 16)] = jnp.zeros((16,), jnp.float32)
        return carry

    lax.fori_loop(0, K, zrow, 0)
    for t in range(ROWS_PER_TILE // K):
        pltpu.async_copy(rows_a,
                         acc_sh.at[pl.ds(s * ROWS_PER_TILE + t * K, K)],
                         ssem_a)
    pltpu.async_copy(src_hbm.at[wid, pl.ds(0, QC)], src_q0, isem_s)
    pltpu.async_copy(dst_hbm.at[wid, pl.ds(0, QC)], dst_q0, isem_d)
    for t in range(ROWS_PER_TILE // K):
        pltpu.make_async_copy(rows_a,
                              acc_sh.at[pl.ds(s * ROWS_PER_TILE + t * K, K)],
                              ssem_a).wait()
    plsc.subcore_barrier()

    bufs = (rows_a, rows_b)
    gsems = (gsem_a, gsem_b)
    ssems = (ssem_a, ssem_b)
    slabs = ((src_q0, dst_q0), (src_q1, dst_q1))

    def slab_fetch(q, p):
        sl = pl.ds(q * QC, QC)
        pltpu.async_copy(src_hbm.at[wid, sl], slabs[p][0], isem_s)
        pltpu.async_copy(dst_hbm.at[wid, sl], slabs[p][1], isem_d)

    def slab_wait(q, p):
        sl = pl.ds(q * QC, QC)
        pltpu.make_async_copy(src_hbm.at[wid, sl], slabs[p][0], isem_s).wait()
        pltpu.make_async_copy(dst_hbm.at[wid, sl], slabs[p][1], isem_d).wait()

    for q in range(5):
        p = q % 2
        src_v, dst_v = slabs[p]
        slab_wait(q, p)
        if q < 4:
            slab_fetch(q + 1, 1 - p)

        def gather(j, b):
            pltpu.async_copy(xs_hbm.at[src_v.at[j]], bufs[b], gsems[b])

        def gather_wait(j, b):
            pltpu.make_async_copy(xs_hbm.at[src_v.at[j]], bufs[b],
                                  gsems[b]).wait()

        def scat(j, b):
            pltpu.async_copy(bufs[b], acc_sh.at[dst_v.at[j]], ssems[b],
                             add=True)

        def scat_wait(j, b):
            pltpu.make_async_copy(bufs[b], acc_sh.at[dst_v.at[j]],
                                  ssems[b]).wait()

        gather(0, 0)
        gather(1, 1)

        def body(i, carry):
            j = 2 * i
            gather_wait(j, 0)
            scat(j, 0)
            gather_wait(j + 1, 1)
            scat(j + 1, 1)
            scat_wait(j, 0)
            gather(j + 2, 0)
            scat_wait(j + 1, 1)
            gather(j + 3, 1)
            return carry

        lax.fori_loop(0, QC // 2 - 1, body, 0)
        j = QC - 2
        gather_wait(j, 0)
        scat(j, 0)
        gather_wait(j + 1, 1)
        scat(j + 1, 1)
        scat_wait(j, 0)
        scat_wait(j + 1, 1)
    plsc.subcore_barrier()

    sl = pl.ds(s * ROWS_PER_TILE, ROWS_PER_TILE)
    pltpu.sync_copy(acc_sh.at[sl], out_hbm.at[c, sl])


def _tc_prep_body(emb_ref, degt_ref, xs_ref):
    deg_out = degt_ref[:, 0:1] + degt_ref[:, 2:3]
    norm_out = lax.rsqrt(jnp.maximum(deg_out, 1.0))
    xs_ref[...] = emb_ref[...] * norm_out


def _tc_final_body(part_ref, degt_ref, out_ref):
    deg_in = degt_ref[:, 1:2] + degt_ref[:, 3:4]
    norm_in = lax.rsqrt(jnp.maximum(deg_in, 1.0))
    out_ref[...] = (part_ref[0] + part_ref[1]) * norm_in


def kernel(embeddings, edge_index):
    pad = jnp.arange(E_PAD - N_EDGES, dtype=jnp.int32) % (N_PAD - N_NODES) \
        + N_NODES
    ei = jnp.concatenate([edge_index, jnp.stack([pad, pad])], axis=1)
    src_r = ei[0].reshape(NW, CHUNKS, K)
    dst_r = ei[1].reshape(NW, CHUNKS, K)
    emb_pad = jnp.pad(embeddings, ((0, N_PAD - N_NODES), (0, 0)))

    mesh = plsc.VectorSubcoreMesh(core_axis_name="c", subcore_axis_name="s")

    bincount = pl.kernel(
        _sc_bincount_body,
        out_type=jax.ShapeDtypeStruct((NC, 2, N_PAD), jnp.float32),
        mesh=mesh,
        scratch_types=[
            pltpu.VMEM((HALFC, K), jnp.int32),
            pltpu.VMEM((HALFC, K), jnp.int32),
            pltpu.VMEM((K,), jnp.float32),
            pltpu.VMEM((ROWS_PER_TILE,), jnp.float32),
            pltpu.VMEM_SHARED((N_PAD,), jnp.float32),
            pltpu.VMEM_SHARED((N_PAD,), jnp.float32),
            pltpu.SemaphoreType.DMA,
            pltpu.SemaphoreType.DMA,
        ],
    )
    deg_part = bincount(src_r, dst_r)
    degt = deg_part.transpose(2, 0, 1).reshape(N_PAD, 4)

    blk = 1024
    grid = N_PAD // blk
    xs = pl.pallas_call(
        _tc_prep_body,
        out_shape=jax.ShapeDtypeStruct((N_PAD, D), jnp.float32),
        grid=(grid,),
        in_specs=[
            pl.BlockSpec((blk, D), lambda i: (i, 0)),
            pl.BlockSpec((blk, 4), lambda i: (i, 0)),
        ],
        out_specs=pl.BlockSpec((blk, D), lambda i: (i, 0)),
    )(emb_pad, degt)

    scatter = pl.kernel(
        _sc_scatter_body,
        out_type=jax.ShapeDtypeStruct((NC, N_PAD, D), jnp.float32),
        mesh=mesh,
        scratch_types=[
            pltpu.VMEM((QC, K), jnp.int32),
            pltpu.VMEM((QC, K), jnp.int32),
            pltpu.VMEM((QC, K), jnp.int32),
            pltpu.VMEM((QC, K), jnp.int32),
            pltpu.VMEM((K, D), jnp.float32),
            pltpu.VMEM((K, D), jnp.float32),
            pltpu.VMEM_SHARED((N_PAD, D), jnp.float32),
            pltpu.SemaphoreType.DMA,
            pltpu.SemaphoreType.DMA,
            pltpu.SemaphoreType.DMA,
            pltpu.SemaphoreType.DMA,
            pltpu.SemaphoreType.DMA,
            pltpu.SemaphoreType.DMA,
        ],
    )
    partials = scatter(xs, src_r, dst_r)

    out = pl.pallas_call(
        _tc_final_body,
        out_shape=jax.ShapeDtypeStruct((N_PAD, D), jnp.float32),
        grid=(grid,),
        in_specs=[
            pl.BlockSpec((NC, blk, D), lambda i: (0, i, 0)),
            pl.BlockSpec((blk, 4), lambda i: (i, 0)),
        ],
        out_specs=pl.BlockSpec((blk, D), lambda i: (i, 0)),
    )(partials, degt)

    return out[:N_NODES]


# speedup vs baseline: 1.1960x; 1.1960x over previous
import functools

import jax
import jax.numpy as jnp
from jax import lax
from jax.experimental import pallas as pl
from jax.experimental.pallas import tpu as pltpu
from jax.experimental.pallas import tpu_sc as plsc

N_NODES = 10000
N_EDGES = 320000
D = 128

NC = 2
NS = 16
NW = NC * NS
K = 128
CHUNKS = 80
EPT = CHUNKS * K
HALFC = CHUNKS // 2
QC = CHUNKS // 5
E_PAD = NW * EPT
N_PAD = 10240
ROWS_PER_TILE = N_PAD // NS


def _sc_bincount_body(src_hbm, dst_hbm, deg_hbm, src_v, dst_v, ones_v, zero_v,
                      deg_src_sh, deg_dst_sh, bsem_s, bsem_d):
    c = lax.axis_index("c")
    s = lax.axis_index("s")
    wid = c * NS + s

    for i in range(ROWS_PER_TILE // 16):
        zero_v[pl.ds(i * 16, 16)] = jnp.zeros((16,), jnp.float32)
    for i in range(K // 16):
        ones_v[pl.ds(i * 16, 16)] = jnp.ones((16,), jnp.float32)
    pltpu.sync_copy(zero_v, deg_src_sh.at[pl.ds(s * ROWS_PER_TILE, ROWS_PER_TILE)])
    pltpu.sync_copy(zero_v, deg_dst_sh.at[pl.ds(s * ROWS_PER_TILE, ROWS_PER_TILE)])

    plsc.subcore_barrier()

    def body(j, carry):
        pltpu.async_copy(ones_v, deg_src_sh.at[src_v.at[j]], bsem_s, add=True)
        pltpu.async_copy(ones_v, deg_dst_sh.at[dst_v.at[j]], bsem_d, add=True)
        pltpu.make_async_copy(ones_v, deg_src_sh.at[src_v.at[j]], bsem_s).wait()
        pltpu.make_async_copy(ones_v, deg_dst_sh.at[dst_v.at[j]], bsem_d).wait()
        return carry

    for phase in range(2):
        pltpu.sync_copy(src_hbm.at[wid, pl.ds(phase * HALFC, HALFC)], src_v)
        pltpu.sync_copy(dst_hbm.at[wid, pl.ds(phase * HALFC, HALFC)], dst_v)
        lax.fori_loop(0, HALFC, body, 0)
    plsc.subcore_barrier()

    sl = pl.ds(s * ROWS_PER_TILE, ROWS_PER_TILE)
    pltpu.sync_copy(deg_src_sh.at[sl], deg_hbm.at[c, 0, sl])
    pltpu.sync_copy(deg_dst_sh.at[sl], deg_hbm.at[c, 1, sl])


def _sc_scatter_body(xs_hbm, src_hbm, dst_hbm, out_hbm, src_q0, dst_q0,
                     src_q1, dst_q1, rows_a, rows_b, acc_sh,
                     gsem_a, gsem_b, ssem_a, ssem_b, isem_s, isem_d):
    c = lax.axis_index("c")
    s = lax.axis_index("s")
    wid = c * NS + s

    def zrow(i, carry):
        for t in range(D // 16):
            rows_a[i, pl.ds(t * 16, 16)] = jnp.zeros((16,), jnp.float32)
        return carry

    lax.fori_loop(0, K, zrow, 0)
    for t in range(ROWS_PER_TILE // K):
        pltpu.async_copy(rows_a,
                         acc_sh.at[pl.ds(s * ROWS_PER_TILE + t * K, K)],
                         ssem_a)
    pltpu.async_copy(src_hbm.at[wid, pl.ds(0, QC)], src_q0, isem_s)
    pltpu.async_copy(dst_hbm.at[wid, pl.ds(0, QC)], dst_q0, isem_d)
    for t in range(ROWS_PER_TILE // K):
        pltpu.make_async_copy(rows_a,
                              acc_sh.at[pl.ds(s * ROWS_PER_TILE + t * K, K)],
                              ssem_a).wait()
    plsc.subcore_barrier()

    bufs = (rows_a, rows_b)
    gsems = (gsem_a, gsem_b)
    ssems = (ssem_a, ssem_b)
    slabs = ((src_q0, dst_q0), (src_q1, dst_q1))

    def slab_fetch(q, p):
        sl = pl.ds(q * QC, QC)
        pltpu.async_copy(src_hbm.at[wid, sl], slabs[p][0], isem_s)
        pltpu.async_copy(dst_hbm.at[wid, sl], slabs[p][1], isem_d)

    def slab_wait(q, p):
        sl = pl.ds(q * QC, QC)
        pltpu.make_async_copy(src_hbm.at[wid, sl], slabs[p][0], isem_s).wait()
        pltpu.make_async_copy(dst_hbm.at[wid, sl], slabs[p][1], isem_d).wait()

    for q in range(5):
        p = q % 2
        src_v, dst_v = slabs[p]
        slab_wait(q, p)
        if q < 4:
            slab_fetch(q + 1, 1 - p)

        def gather(j, b):
            pltpu.async_copy(xs_hbm.at[src_v.at[j]], bufs[b], gsems[b])

        def gather_wait(j, b):
            pltpu.make_async_copy(xs_hbm.at[src_v.at[j]], bufs[b],
                                  gsems[b]).wait()

        def scat(j, b):
            pltpu.async_copy(bufs[b], acc_sh.at[dst_v.at[j]], ssems[b],
                             add=True)

        def scat_wait(j, b):
            pltpu.make_async_copy(bufs[b], acc_sh.at[dst_v.at[j]],
                                  ssems[b]).wait()

        gather(0, 0)
        gather(1, 1)

        def body(i, carry):
            j = 2 * i
            gather_wait(j, 0)
            scat(j, 0)
            gather_wait(j + 1, 1)
            scat_wait(j, 0)
            gather(j + 2, 0)
            scat(j + 1, 1)
            scat_wait(j + 1, 1)
            gather(j + 3, 1)
            return carry

        lax.fori_loop(0, QC // 2 - 1, body, 0)
        j = QC - 2
        gather_wait(j, 0)
        scat(j, 0)
        gather_wait(j + 1, 1)
        scat_wait(j, 0)
        scat(j + 1, 1)
        scat_wait(j + 1, 1)
    plsc.subcore_barrier()

    sl = pl.ds(s * ROWS_PER_TILE, ROWS_PER_TILE)
    pltpu.sync_copy(acc_sh.at[sl], out_hbm.at[c, sl])


def _tc_prep_body(emb_ref, degt_ref, xs_ref):
    deg_out = degt_ref[:, 0:1] + degt_ref[:, 2:3]
    norm_out = lax.rsqrt(jnp.maximum(deg_out, 1.0))
    xs_ref[...] = emb_ref[...] * norm_out


def _tc_final_body(part_ref, degt_ref, out_ref):
    deg_in = degt_ref[:, 1:2] + degt_ref[:, 3:4]
    norm_in = lax.rsqrt(jnp.maximum(deg_in, 1.0))
    out_ref[...] = (part_ref[0] + part_ref[1]) * norm_in


def kernel(embeddings, edge_index):
    pad = jnp.arange(E_PAD - N_EDGES, dtype=jnp.int32) % (N_PAD - N_NODES) \
        + N_NODES
    ei = jnp.concatenate([edge_index, jnp.stack([pad, pad])], axis=1)
    src_r = ei[0].reshape(NW, CHUNKS, K)
    dst_r = ei[1].reshape(NW, CHUNKS, K)

    mesh = plsc.VectorSubcoreMesh(core_axis_name="c", subcore_axis_name="s")

    bincount = pl.kernel(
        _sc_bincount_body,
        out_type=jax.ShapeDtypeStruct((NC, 2, N_PAD), jnp.float32),
        mesh=mesh,
        scratch_types=[
            pltpu.VMEM((HALFC, K), jnp.int32),
            pltpu.VMEM((HALFC, K), jnp.int32),
            pltpu.VMEM((K,), jnp.float32),
            pltpu.VMEM((ROWS_PER_TILE,), jnp.float32),
            pltpu.VMEM_SHARED((N_PAD,), jnp.float32),
            pltpu.VMEM_SHARED((N_PAD,), jnp.float32),
            pltpu.SemaphoreType.DMA,
            pltpu.SemaphoreType.DMA,
        ],
    )
    deg_part = bincount(src_r, dst_r)
    degt = deg_part.transpose(2, 0, 1).reshape(N_PAD, 4)

    blk = 1000
    grid = N_NODES // blk
    xs = pl.pallas_call(
        _tc_prep_body,
        out_shape=jax.ShapeDtypeStruct((N_PAD, D), jnp.float32),
        grid=(grid,),
        in_specs=[
            pl.BlockSpec((blk, D), lambda i: (i, 0)),
            pl.BlockSpec((blk, 4), lambda i: (i, 0)),
        ],
        out_specs=pl.BlockSpec((blk, D), lambda i: (i, 0)),
    )(embeddings, degt)

    scatter = pl.kernel(
        _sc_scatter_body,
        out_type=jax.ShapeDtypeStruct((NC, N_PAD, D), jnp.float32),
        mesh=mesh,
        scratch_types=[
            pltpu.VMEM((QC, K), jnp.int32),
            pltpu.VMEM((QC, K), jnp.int32),
            pltpu.VMEM((QC, K), jnp.int32),
            pltpu.VMEM((QC, K), jnp.int32),
            pltpu.VMEM((K, D), jnp.float32),
            pltpu.VMEM((K, D), jnp.float32),
            pltpu.VMEM_SHARED((N_PAD, D), jnp.float32),
            pltpu.SemaphoreType.DMA,
            pltpu.SemaphoreType.DMA,
            pltpu.SemaphoreType.DMA,
            pltpu.SemaphoreType.DMA,
            pltpu.SemaphoreType.DMA,
            pltpu.SemaphoreType.DMA,
        ],
    )
    partials = scatter(xs, src_r, dst_r)

    out = pl.pallas_call(
        _tc_final_body,
        out_shape=jax.ShapeDtypeStruct((N_NODES, D), jnp.float32),
        grid=(grid,),
        in_specs=[
            pl.BlockSpec((NC, blk, D), lambda i: (0, i, 0)),
            pl.BlockSpec((blk, 4), lambda i: (i, 0)),
        ],
        out_specs=pl.BlockSpec((blk, D), lambda i: (i, 0)),
    )(partials, degt)

    return out


# speedup vs baseline: 1.1974x; 1.0012x over previous
import jax
import jax.numpy as jnp
from jax import lax
from jax.experimental import pallas as pl
from jax.experimental.pallas import tpu as pltpu
from jax.experimental.pallas import tpu_sc as plsc

N_NODES = 10000
N_EDGES = 320000
D = 128

NC = 2
NS = 16
NW = NC * NS
K = 128
CHUNKS = 80
EPT = CHUNKS * K
HALFC = CHUNKS // 2
QC = CHUNKS // 5
E_PAD = NW * EPT
N_PAD = 10240
ROWS_PER_TILE = N_PAD // NS


def _sc_bincount_body(src_hbm, dst_hbm, deg_hbm, src_v, dst_v, ones_v, zero_v,
                      deg_src_sh, deg_dst_sh, bsem_s, bsem_d):
    c = lax.axis_index("c")
    s = lax.axis_index("s")
    wid = c * NS + s

    for i in range(ROWS_PER_TILE // 16):
        zero_v[pl.ds(i * 16, 16)] = jnp.zeros((16,), jnp.float32)
    for i in range(K // 16):
        ones_v[pl.ds(i * 16, 16)] = jnp.ones((16,), jnp.float32)
    pltpu.sync_copy(zero_v, deg_src_sh.at[pl.ds(s * ROWS_PER_TILE, ROWS_PER_TILE)])
    pltpu.sync_copy(zero_v, deg_dst_sh.at[pl.ds(s * ROWS_PER_TILE, ROWS_PER_TILE)])

    plsc.subcore_barrier()

    def body(j, carry):
        pltpu.async_copy(ones_v, deg_src_sh.at[src_v.at[j]], bsem_s, add=True)
        pltpu.async_copy(ones_v, deg_dst_sh.at[dst_v.at[j]], bsem_d, add=True)
        pltpu.make_async_copy(ones_v, deg_src_sh.at[src_v.at[j]], bsem_s).wait()
        pltpu.make_async_copy(ones_v, deg_dst_sh.at[dst_v.at[j]], bsem_d).wait()
        return carry

    for phase in range(2):
        pltpu.sync_copy(src_hbm.at[wid, pl.ds(phase * HALFC, HALFC)], src_v)
        pltpu.sync_copy(dst_hbm.at[wid, pl.ds(phase * HALFC, HALFC)], dst_v)
        lax.fori_loop(0, HALFC, body, 0)
    plsc.subcore_barrier()

    sl = pl.ds(s * ROWS_PER_TILE, ROWS_PER_TILE)
    pltpu.sync_copy(deg_src_sh.at[sl], deg_hbm.at[c, 0, sl])
    pltpu.sync_copy(deg_dst_sh.at[sl], deg_hbm.at[c, 1, sl])


def _sc_scatter_body(xs_hbm, src_hbm, dst_hbm, out_hbm, src_q0, dst_q0,
                     src_q1, dst_q1, rows_a, rows_b, acc_sh,
                     gsem_a, gsem_b, ssem_a, ssem_b, isem_s, isem_d):
    c = lax.axis_index("c")
    s = lax.axis_index("s")
    wid = c * NS + s

    def zrow(i, carry):
        for t in range(D // 16):
            rows_a[i, pl.ds(t * 16, 16)] = jnp.zeros((16,), jnp.float32)
        return carry

    lax.fori_loop(0, K, zrow, 0)
    for t in range(ROWS_PER_TILE // K):
        pltpu.async_copy(rows_a,
                         acc_sh.at[pl.ds(s * ROWS_PER_TILE + t * K, K)],
                         ssem_a)
    pltpu.async_copy(src_hbm.at[wid, pl.ds(0, QC)], src_q0, isem_s)
    pltpu.async_copy(dst_hbm.at[wid, pl.ds(0, QC)], dst_q0, isem_d)
    for t in range(ROWS_PER_TILE // K):
        pltpu.make_async_copy(rows_a,
                              acc_sh.at[pl.ds(s * ROWS_PER_TILE + t * K, K)],
                              ssem_a).wait()
    plsc.subcore_barrier()

    bufs = (rows_a, rows_b)
    gsems = (gsem_a, gsem_b)
    ssems = (ssem_a, ssem_b)
    slabs = ((src_q0, dst_q0), (src_q1, dst_q1))

    def slab_fetch(q, p):
        sl = pl.ds(q * QC, QC)
        pltpu.async_copy(src_hbm.at[wid, sl], slabs[p][0], isem_s)
        pltpu.async_copy(dst_hbm.at[wid, sl], slabs[p][1], isem_d)

    def slab_wait(q, p):
        sl = pl.ds(q * QC, QC)
        pltpu.make_async_copy(src_hbm.at[wid, sl], slabs[p][0], isem_s).wait()
        pltpu.make_async_copy(dst_hbm.at[wid, sl], slabs[p][1], isem_d).wait()

    for q in range(5):
        p = q % 2
        src_v, dst_v = slabs[p]
        slab_wait(q, p)
        if q < 4:
            slab_fetch(q + 1, 1 - p)

        def gather(j, b):
            pltpu.async_copy(xs_hbm.at[src_v.at[j]], bufs[b], gsems[b])

        def gather_wait(j, b):
            pltpu.make_async_copy(xs_hbm.at[src_v.at[j]], bufs[b],
                                  gsems[b]).wait()

        def scat(j, b):
            pltpu.async_copy(bufs[b], acc_sh.at[dst_v.at[j]], ssems[b],
                             add=True)

        def scat_wait(j, b):
            pltpu.make_async_copy(bufs[b], acc_sh.at[dst_v.at[j]],
                                  ssems[b]).wait()

        gather(0, 0)
        gather(1, 1)

        def body(i, carry):
            j = 2 * i
            gather_wait(j, 0)
            scat(j, 0)
            gather_wait(j + 1, 1)
            scat_wait(j, 0)
            gather(j + 2, 0)
            scat(j + 1, 1)
            scat_wait(j + 1, 1)
            gather(j + 3, 1)
            return carry

        lax.fori_loop(0, QC // 2 - 1, body, 0)
        j = QC - 2
        gather_wait(j, 0)
        scat(j, 0)
        gather_wait(j + 1, 1)
        scat_wait(j, 0)
        scat(j + 1, 1)
        scat_wait(j + 1, 1)
    plsc.subcore_barrier()

    sl = pl.ds(s * ROWS_PER_TILE, ROWS_PER_TILE)
    pltpu.sync_copy(acc_sh.at[sl], out_hbm.at[c, sl])


def _tc_prep_body(emb_ref, degt_ref, xs_ref):
    deg_out = degt_ref[:, 0:1] + degt_ref[:, 2:3]
    norm_out = lax.rsqrt(jnp.maximum(deg_out, 1.0))
    xs_ref[...] = emb_ref[...] * norm_out


def _tc_final_body(part_ref, degt_ref, out_ref):
    deg_in = degt_ref[:, 1:2] + degt_ref[:, 3:4]
    norm_in = lax.rsqrt(jnp.maximum(deg_in, 1.0))
    out_ref[...] = (part_ref[0] + part_ref[1]) * norm_in


def kernel(embeddings, edge_index):
    pad = jnp.arange(E_PAD - N_EDGES, dtype=jnp.int32) % (N_PAD - N_NODES) \
        + N_NODES
    ei = jnp.concatenate([edge_index, jnp.stack([pad, pad])], axis=1)
    src_r = ei[0].reshape(NW, CHUNKS, K)
    dst_r = ei[1].reshape(NW, CHUNKS, K)

    mesh = plsc.VectorSubcoreMesh(core_axis_name="c", subcore_axis_name="s")

    bincount = pl.kernel(
        _sc_bincount_body,
        out_type=jax.ShapeDtypeStruct((NC, 2, N_PAD), jnp.float32),
        mesh=mesh,
        scratch_types=[
            pltpu.VMEM((HALFC, K), jnp.int32),
            pltpu.VMEM((HALFC, K), jnp.int32),
            pltpu.VMEM((K,), jnp.float32),
            pltpu.VMEM((ROWS_PER_TILE,), jnp.float32),
            pltpu.VMEM_SHARED((N_PAD,), jnp.float32),
            pltpu.VMEM_SHARED((N_PAD,), jnp.float32),
            pltpu.SemaphoreType.DMA,
            pltpu.SemaphoreType.DMA,
        ],
    )
    deg_part = bincount(src_r, dst_r)
    degt = deg_part.transpose(2, 0, 1).reshape(N_PAD, 4)

    blk = 1000
    grid = N_NODES // blk
    xs = pl.pallas_call(
        _tc_prep_body,
        out_shape=jax.ShapeDtypeStruct((N_PAD, D), jnp.float32),
        grid=(grid,),
        in_specs=[
            pl.BlockSpec((blk, D), lambda i: (i, 0)),
            pl.BlockSpec((blk, 4), lambda i: (i, 0)),
        ],
        out_specs=pl.BlockSpec((blk, D), lambda i: (i, 0)),
    )(embeddings, degt)

    scatter = pl.kernel(
        _sc_scatter_body,
        out_type=jax.ShapeDtypeStruct((NC, N_PAD, D), jnp.float32),
        mesh=mesh,
        scratch_types=[
            pltpu.VMEM((QC, K), jnp.int32),
            pltpu.VMEM((QC, K), jnp.int32),
            pltpu.VMEM((QC, K), jnp.int32),
            pltpu.VMEM((QC, K), jnp.int32),
            pltpu.VMEM((K, D), jnp.float32),
            pltpu.VMEM((K, D), jnp.float32),
            pltpu.VMEM_SHARED((N_PAD, D), jnp.float32),
            pltpu.SemaphoreType.DMA,
            pltpu.SemaphoreType.DMA,
            pltpu.SemaphoreType.DMA,
            pltpu.SemaphoreType.DMA,
            pltpu.SemaphoreType.DMA,
            pltpu.SemaphoreType.DMA,
        ],
    )
    partials = scatter(xs, src_r, dst_r)

    out = pl.pallas_call(
        _tc_final_body,
        out_shape=jax.ShapeDtypeStruct((N_NODES, D), jnp.float32),
        grid=(grid,),
        in_specs=[
            pl.BlockSpec((NC, blk, D), lambda i: (0, i, 0)),
            pl.BlockSpec((blk, 4), lambda i: (i, 0)),
        ],
        out_specs=pl.BlockSpec((blk, D), lambda i: (i, 0)),
    )(partials, degt)

    return out
